# baseline (speedup 1.0000x reference)
import functools

import jax
import jax.numpy as jnp
from jax import lax
from jax.experimental import pallas as pl
from jax.experimental.pallas import tpu as pltpu
from jax.experimental.pallas import tpu_sc as plsc

_CH = 128


@functools.cache
def _build(B, D):
    info = plsc.get_sparse_core_info()
    NC, NS = info.num_cores, info.num_subcores
    NW = NC * NS
    b_per_w = B // NW
    n_chunks = b_per_w // _CH
    mesh = plsc.VectorSubcoreMesh(core_axis_name="c", subcore_axis_name="s")

    @functools.partial(
        pl.kernel,
        out_type=jax.ShapeDtypeStruct((B, D), jnp.float32),
        mesh=mesh,
        scratch_types=[
            pltpu.VMEM((n_chunks, _CH), jnp.int32),
            pltpu.VMEM((_CH, D), jnp.float32),
            pltpu.SemaphoreType.DMA,
        ],
        compiler_params=pltpu.CompilerParams(use_tc_tiling_on_sc=False),
    )
    def gather_kernel(idx_hbm, table_hbm, out_hbm, idx_v, rows_v, sem):
        wid = lax.axis_index("s") * NC + lax.axis_index("c")
        base_row = wid * n_chunks
        pltpu.sync_copy(idx_hbm.at[pl.ds(base_row, n_chunks)], idx_v)

        def body(j, carry):
            pltpu.async_copy(table_hbm.at[idx_v.at[j]], rows_v, sem).wait()
            pltpu.sync_copy(rows_v, out_hbm.at[pl.ds((base_row + j) * _CH, _CH)])
            return carry

        lax.fori_loop(0, n_chunks, body, 0)

    return gather_kernel


def kernel(x, table):
    S0, S1 = x.shape
    B = S0 * S1
    D = table.shape[1]
    idx = x.reshape(B // _CH, _CH).astype(jnp.int32)
    out = _build(B, D)(idx, table)
    return out.reshape(S0, S1, D)


# speedup vs baseline: 1.1141x; 1.1141x over previous
import functools

import jax
import jax.numpy as jnp
from jax import lax
from jax.experimental import pallas as pl
from jax.experimental.pallas import tpu as pltpu
from jax.experimental.pallas import tpu_sc as plsc

_CH = 128
_NBUF = 8
_S = _NBUF // 2


@functools.cache
def _build(B, D):
    info = plsc.get_sparse_core_info()
    NC, NS = info.num_cores, info.num_subcores
    NW = NC * NS
    b_per_w = B // NW
    n_chunks = b_per_w // _CH
    n_groups = n_chunks // _NBUF
    assert B % NW == 0 and b_per_w % _CH == 0 and n_chunks % _NBUF == 0
    assert n_groups >= 2
    mesh = plsc.VectorSubcoreMesh(core_axis_name="c", subcore_axis_name="s")

    @functools.partial(
        pl.kernel,
        out_type=jax.ShapeDtypeStruct((B, D), jnp.float32),
        mesh=mesh,
        scratch_types=[
            pltpu.VMEM((n_chunks, _CH), jnp.int32),
            pltpu.VMEM((_NBUF, _CH, D), jnp.float32),
            pltpu.SemaphoreType.DMA,
            pltpu.SemaphoreType.DMA,
        ],
        compiler_params=pltpu.CompilerParams(use_tc_tiling_on_sc=False),
    )
    def gather_kernel(idx_hbm, table_hbm, out_hbm, idx_v, rows_v, gsem, ssem):
        wid = lax.axis_index("s") * NC + lax.axis_index("c")
        base_row = wid * n_chunks
        pltpu.sync_copy(idx_hbm.at[pl.ds(base_row, n_chunks)], idx_v)

        def start_gather(j, b):
            pltpu.async_copy(table_hbm.at[idx_v.at[j]], rows_v.at[b], gsem)

        def wait_gather(b):
            pltpu.make_async_copy(
                table_hbm.at[pl.ds(0, _CH)], rows_v.at[b], gsem).wait()

        def start_store(j, b):
            pltpu.async_copy(
                rows_v.at[b], out_hbm.at[pl.ds((base_row + j) * _CH, _CH)], ssem)

        def wait_store(b):
            pltpu.make_async_copy(
                rows_v.at[b], out_hbm.at[pl.ds(0, _CH)], ssem).wait()

        for b in range(_NBUF):
            start_gather(b, b)

        for b in range(_NBUF):
            wait_gather(b)
            start_store(b, b)
            if b >= _S:
                wait_store(b - _S)
                start_gather(b + _S, b - _S)

        def group(g, carry):
            j0 = g * _NBUF
            for b in range(_NBUF):
                j = j0 + b
                wait_gather(b)
                start_store(j, b)
                wait_store((b + _S) % _NBUF)
                start_gather(j + _S, (b + _S) % _NBUF)
            return carry

        lax.fori_loop(1, n_groups - 1, group, 0)

        j0 = (n_groups - 1) * _NBUF
        for b in range(_NBUF):
            j = j0 + b
            wait_gather(b)
            start_store(j, b)
            wait_store((b + _S) % _NBUF)
            if b < _S:
                start_gather(j + _S, (b + _S) % _NBUF)
        for b in range(_S):
            wait_store(b)

    return gather_kernel


def kernel(x, table):
    S0, S1 = x.shape
    B = S0 * S1
    D = table.shape[1]
    idx = x.reshape(B // _CH, _CH).astype(jnp.int32)
    out = _build(B, D)(idx, table)
    return out.reshape(S0, S1, D)
